# Initial kernel scaffold; baseline (speedup 1.0000x reference)
import jax
import jax.numpy as jnp
from jax.experimental import pallas as pl


def kernel(x, edge_index_r0, edge_index_r1, W0_r0, b0_r0, W0_r1, b0_r1, W1_r0, b1_r0, W1_r1, b1_r1, ln0_g, ln0_b, ln1_g, ln1_b):
    raise NotImplementedError("write your pallas kernel here")


# speedup vs baseline: 21.2367x; 21.2367x over previous
import functools

import jax
import jax.numpy as jnp
from jax import lax
from jax.experimental import pallas as pl
from jax.experimental.pallas import tpu as pltpu
from jax.experimental.pallas import tpu_sc as plsc

N = 10000
E = 160000
D = 128
NP = 10240
NT = 16
ROWS_PER_TILE = NP // NT
K = 80
NCH = (E // NT) // K


def _deg_kernel_body(dst_hbm, zero1_hbm, deg_hbm, dst_v, ones_v, hist_sh):
    c = lax.axis_index("c")
    s = lax.axis_index("s")
    wid = c * NT + s
    pltpu.sync_copy(dst_hbm.at[wid], dst_v)
    pltpu.sync_copy(zero1_hbm.at[pl.ds(s * ROWS_PER_TILE, ROWS_PER_TILE)],
                    hist_sh.at[pl.ds(s * ROWS_PER_TILE, ROWS_PER_TILE)])
    for i in range(K // 16):
        ones_v[pl.ds(i * 16, 16)] = jnp.ones((16,), jnp.float32)
    plsc.subcore_barrier()

    def body(j, carry):
        pltpu.sync_copy(ones_v, hist_sh.at[dst_v.at[j]], add=True)
        return carry

    lax.fori_loop(0, NCH, body, 0)
    plsc.subcore_barrier()
    pltpu.sync_copy(hist_sh.at[pl.ds(s * ROWS_PER_TILE, ROWS_PER_TILE)],
                    deg_hbm.at[c, pl.ds(s * ROWS_PER_TILE, ROWS_PER_TILE)])


def _scatter_kernel_body(hcat_hbm, ed_hbm, zrows_hbm, out_hbm,
                         idx0, idx1, rows0, rows1, acc_sh, sem0, sem1):
    c = lax.axis_index("c")
    s = lax.axis_index("s")
    wid = c * NT + s
    pltpu.sync_copy(zrows_hbm.at[pl.ds(s * ROWS_PER_TILE, ROWS_PER_TILE)],
                    acc_sh.at[pl.ds(s * ROWS_PER_TILE, ROWS_PER_TILE)])
    plsc.subcore_barrier()

    pltpu.sync_copy(ed_hbm.at[wid, 0], idx0)
    pltpu.async_copy(hcat_hbm.at[idx0.at[0]], rows0, sem0)

    def body(p, carry):
        j0 = 2 * p
        pltpu.sync_copy(ed_hbm.at[wid, j0 + 1], idx1)
        pltpu.make_async_copy(hcat_hbm.at[idx0.at[0]], rows0, sem0).wait()
        pltpu.async_copy(hcat_hbm.at[idx1.at[0]], rows1, sem1)
        pltpu.sync_copy(rows0, acc_sh.at[idx0.at[1]], add=True)
        pltpu.sync_copy(ed_hbm.at[wid, j0 + 2], idx0)
        pltpu.make_async_copy(hcat_hbm.at[idx1.at[0]], rows1, sem1).wait()
        pltpu.async_copy(hcat_hbm.at[idx0.at[0]], rows0, sem0)
        pltpu.sync_copy(rows1, acc_sh.at[idx1.at[1]], add=True)
        return carry

    lax.fori_loop(0, (NCH - 1) // 2, body, 0)
    pltpu.make_async_copy(hcat_hbm.at[idx0.at[0]], rows0, sem0).wait()
    pltpu.sync_copy(rows0, acc_sh.at[idx0.at[1]], add=True)

    plsc.subcore_barrier()
    pltpu.sync_copy(acc_sh.at[pl.ds(s * ROWS_PER_TILE, ROWS_PER_TILE)],
                    out_hbm.at[c, pl.ds(s * ROWS_PER_TILE, ROWS_PER_TILE)])


@functools.lru_cache(maxsize=None)
def _sc_kernels():
    mesh = plsc.VectorSubcoreMesh(core_axis_name="c", subcore_axis_name="s")
    deg_k = pl.kernel(
        _deg_kernel_body,
        out_type=jax.ShapeDtypeStruct((2, NP), jnp.float32),
        mesh=mesh,
        scratch_types=[
            pltpu.VMEM((NCH, K), jnp.int32),
            pltpu.VMEM((K,), jnp.float32),
            pltpu.VMEM_SHARED((NP,), jnp.float32),
        ],
    )
    scatter_k = pl.kernel(
        _scatter_kernel_body,
        out_type=jax.ShapeDtypeStruct((2, NP, D), jnp.float32),
        mesh=mesh,
        scratch_types=[
            pltpu.VMEM((2, K), jnp.int32),
            pltpu.VMEM((2, K), jnp.int32),
            pltpu.VMEM((K, D), jnp.float32),
            pltpu.VMEM((K, D), jnp.float32),
            pltpu.VMEM_SHARED((NP, D), jnp.float32),
            pltpu.SemaphoreType.DMA,
            pltpu.SemaphoreType.DMA,
        ],
    )
    return deg_k, scatter_k


_BM = 1024
_GRID = NP // _BM


def _prep_body(x_ref, w0_ref, w1_ref, deg_ref, hcat_ref, dinv_ref):
    dv = lax.rsqrt(deg_ref[...] + 1.0)
    dinv_ref[...] = dv
    xb = x_ref[...]
    hcat_ref[0] = jnp.dot(xb, w0_ref[...],
                          preferred_element_type=jnp.float32) * dv[0][:, None]
    hcat_ref[1] = jnp.dot(xb, w1_ref[...],
                          preferred_element_type=jnp.float32) * dv[1][:, None]


def _combine(out_ref, hcat_ref, dinv_ref, b0_ref, b1_ref, g_ref, beta_ref):
    dv = dinv_ref[...]
    sm = (out_ref[0] + hcat_ref[0]) * dv[0][:, None] + b0_ref[...]
    sm = sm + (out_ref[1] + hcat_ref[1]) * dv[1][:, None] + b1_ref[...]
    mu = jnp.mean(sm, axis=1, keepdims=True)
    var = jnp.mean((sm - mu) ** 2, axis=1, keepdims=True)
    hn = (sm - mu) * lax.rsqrt(var + 1e-5) * g_ref[...] + beta_ref[...]
    return jnp.maximum(hn, 0.0), dv


def _mid_body(out_ref, hcat_ref, dinv_ref, b0_ref, b1_ref, g_ref, beta_ref,
              w0_ref, w1_ref, hcat2_ref):
    h, dv = _combine(out_ref, hcat_ref, dinv_ref, b0_ref, b1_ref, g_ref, beta_ref)
    hcat2_ref[0] = jnp.dot(h, w0_ref[...],
                           preferred_element_type=jnp.float32) * dv[0][:, None]
    hcat2_ref[1] = jnp.dot(h, w1_ref[...],
                           preferred_element_type=jnp.float32) * dv[1][:, None]


def _final_body(out_ref, hcat_ref, dinv_ref, b0_ref, b1_ref, g_ref, beta_ref,
                h_ref):
    h, _ = _combine(out_ref, hcat_ref, dinv_ref, b0_ref, b1_ref, g_ref, beta_ref)
    h_ref[...] = h


_spec_nodes = pl.BlockSpec((_BM, D), lambda i: (i, 0))
_spec_w = pl.BlockSpec((D, D), lambda i: (0, 0))
_spec_vecD = pl.BlockSpec((1, D), lambda i: (0, 0))
_spec_2n = pl.BlockSpec((2, _BM), lambda i: (0, i))
_spec_2nd = pl.BlockSpec((2, _BM, D), lambda i: (0, i, 0))

_prep_call = pl.pallas_call(
    _prep_body,
    grid=(_GRID,),
    in_specs=[_spec_nodes, _spec_w, _spec_w, _spec_2n],
    out_specs=[_spec_2nd, _spec_2n],
    out_shape=[jax.ShapeDtypeStruct((2, NP, D), jnp.float32),
               jax.ShapeDtypeStruct((2, NP), jnp.float32)],
)

_mid_call = pl.pallas_call(
    _mid_body,
    grid=(_GRID,),
    in_specs=[_spec_2nd, _spec_2nd, _spec_2n, _spec_vecD, _spec_vecD,
              _spec_vecD, _spec_vecD, _spec_w, _spec_w],
    out_specs=_spec_2nd,
    out_shape=jax.ShapeDtypeStruct((2, NP, D), jnp.float32),
)

_final_call = pl.pallas_call(
    _final_body,
    grid=(_GRID,),
    in_specs=[_spec_2nd, _spec_2nd, _spec_2n, _spec_vecD, _spec_vecD,
              _spec_vecD, _spec_vecD],
    out_specs=_spec_nodes,
    out_shape=jax.ShapeDtypeStruct((NP, D), jnp.float32),
)


def kernel(x, edge_index_r0, edge_index_r1, W0_r0, b0_r0, W0_r1, b0_r1,
           W1_r0, b1_r0, W1_r1, b1_r1, ln0_g, ln0_b, ln1_g, ln1_b):
    f32 = jnp.float32
    src_all = jnp.concatenate(
        [edge_index_r0[0], edge_index_r1[0] + NP]).reshape(2 * NT, NCH, 1, K)
    dst_all = jnp.concatenate(
        [edge_index_r0[1], edge_index_r1[1]]).reshape(2 * NT, NCH, 1, K)
    ed_all = jnp.concatenate([src_all, dst_all], axis=2)
    zero1 = jnp.zeros((NP,), f32)
    zrows = jnp.zeros((NP, D), f32)
    x_pad = jnp.pad(x, ((0, NP - N), (0, 0)))

    _deg_kernel, _scatter_kernel = _sc_kernels()
    deg = _deg_kernel(dst_all.reshape(2 * NT, NCH, K), zero1)

    hcat1, dinv = _prep_call(x_pad, W0_r0, W0_r1, deg)
    out1 = _scatter_kernel(hcat1.reshape(2 * NP, D), ed_all, zrows)
    hcat2 = _mid_call(out1, hcat1, dinv,
                      b0_r0.reshape(1, D), b0_r1.reshape(1, D),
                      ln0_g.reshape(1, D), ln0_b.reshape(1, D), W1_r0, W1_r1)
    out2 = _scatter_kernel(hcat2.reshape(2 * NP, D), ed_all, zrows)
    h = _final_call(out2, hcat2, dinv,
                    b1_r0.reshape(1, D), b1_r1.reshape(1, D),
                    ln1_g.reshape(1, D), ln1_b.reshape(1, D))
    return h[:N]


# speedup vs baseline: 24.6438x; 1.1604x over previous
import functools

import jax
import jax.numpy as jnp
from jax import lax
from jax.experimental import pallas as pl
from jax.experimental.pallas import tpu as pltpu
from jax.experimental.pallas import tpu_sc as plsc

N = 10000
E = 160000
D = 128
NP = 10240
NT = 16
ROWS_PER_TILE = NP // NT
K = 125
NCH = (E // NT) // K
K_DEG = 80
NCH_DEG = (E // NT) // K_DEG


def _deg_kernel_body(dst_hbm, zero1_hbm, deg_hbm, dst_v, ones_v, hist_sh):
    c = lax.axis_index("c")
    s = lax.axis_index("s")
    wid = c * NT + s
    pltpu.sync_copy(dst_hbm.at[wid], dst_v)
    pltpu.sync_copy(zero1_hbm.at[pl.ds(s * ROWS_PER_TILE, ROWS_PER_TILE)],
                    hist_sh.at[pl.ds(s * ROWS_PER_TILE, ROWS_PER_TILE)])
    for i in range(K_DEG // 16):
        ones_v[pl.ds(i * 16, 16)] = jnp.ones((16,), jnp.float32)
    plsc.subcore_barrier()

    def body(j, carry):
        pltpu.sync_copy(ones_v, hist_sh.at[dst_v.at[j]], add=True)
        return carry

    lax.fori_loop(0, NCH_DEG, body, 0)
    plsc.subcore_barrier()
    pltpu.sync_copy(hist_sh.at[pl.ds(s * ROWS_PER_TILE, ROWS_PER_TILE)],
                    deg_hbm.at[c, pl.ds(s * ROWS_PER_TILE, ROWS_PER_TILE)])


def _scatter_kernel_body(hcat_hbm, ed_hbm, zrows_hbm, out_hbm,
                         idx0, idx1, rows0, rows1, acc_sh, sem0, sem1):
    c = lax.axis_index("c")
    s = lax.axis_index("s")
    wid = c * NT + s
    pltpu.sync_copy(zrows_hbm.at[pl.ds(s * ROWS_PER_TILE, ROWS_PER_TILE)],
                    acc_sh.at[pl.ds(s * ROWS_PER_TILE, ROWS_PER_TILE)])
    plsc.subcore_barrier()

    pltpu.sync_copy(ed_hbm.at[wid, 0], idx0)
    pltpu.async_copy(hcat_hbm.at[idx0.at[0]], rows0, sem0)

    def body(p, carry):
        j0 = 2 * p
        pltpu.sync_copy(ed_hbm.at[wid, j0 + 1], idx1)
        pltpu.make_async_copy(hcat_hbm.at[idx0.at[0]], rows0, sem0).wait()
        pltpu.async_copy(hcat_hbm.at[idx1.at[0]], rows1, sem1)
        pltpu.sync_copy(rows0, acc_sh.at[idx0.at[1]], add=True)
        pltpu.sync_copy(ed_hbm.at[wid, j0 + 2], idx0)
        pltpu.make_async_copy(hcat_hbm.at[idx1.at[0]], rows1, sem1).wait()
        pltpu.async_copy(hcat_hbm.at[idx0.at[0]], rows0, sem0)
        pltpu.sync_copy(rows1, acc_sh.at[idx1.at[1]], add=True)
        return carry

    lax.fori_loop(0, NCH // 2 - 1, body, 0)
    pltpu.sync_copy(ed_hbm.at[wid, NCH - 1], idx1)
    pltpu.make_async_copy(hcat_hbm.at[idx0.at[0]], rows0, sem0).wait()
    pltpu.async_copy(hcat_hbm.at[idx1.at[0]], rows1, sem1)
    pltpu.sync_copy(rows0, acc_sh.at[idx0.at[1]], add=True)
    pltpu.make_async_copy(hcat_hbm.at[idx1.at[0]], rows1, sem1).wait()
    pltpu.sync_copy(rows1, acc_sh.at[idx1.at[1]], add=True)

    plsc.subcore_barrier()
    pltpu.sync_copy(acc_sh.at[pl.ds(s * ROWS_PER_TILE, ROWS_PER_TILE)],
                    out_hbm.at[c, pl.ds(s * ROWS_PER_TILE, ROWS_PER_TILE)])


@functools.lru_cache(maxsize=None)
def _sc_kernels():
    mesh = plsc.VectorSubcoreMesh(core_axis_name="c", subcore_axis_name="s")
    deg_k = pl.kernel(
        _deg_kernel_body,
        out_type=jax.ShapeDtypeStruct((2, NP), jnp.float32),
        mesh=mesh,
        scratch_types=[
            pltpu.VMEM((NCH_DEG, K_DEG), jnp.int32),
            pltpu.VMEM((K_DEG,), jnp.float32),
            pltpu.VMEM_SHARED((NP,), jnp.float32),
        ],
    )
    scatter_k = pl.kernel(
        _scatter_kernel_body,
        out_type=jax.ShapeDtypeStruct((2, NP, D), jnp.float32),
        mesh=mesh,
        scratch_types=[
            pltpu.VMEM((2, K), jnp.int32),
            pltpu.VMEM((2, K), jnp.int32),
            pltpu.VMEM((K, D), jnp.float32),
            pltpu.VMEM((K, D), jnp.float32),
            pltpu.VMEM_SHARED((NP, D), jnp.float32),
            pltpu.SemaphoreType.DMA,
            pltpu.SemaphoreType.DMA,
        ],
    )
    return deg_k, scatter_k


_BM = 1024
_GRID = NP // _BM


def _prep_body(x_ref, w0_ref, w1_ref, deg_ref, hcat_ref, dinv_ref):
    dv = lax.rsqrt(deg_ref[...] + 1.0)
    dinv_ref[...] = dv
    xb = x_ref[...]
    hcat_ref[0] = jnp.dot(xb, w0_ref[...],
                          preferred_element_type=jnp.float32) * dv[0][:, None]
    hcat_ref[1] = jnp.dot(xb, w1_ref[...],
                          preferred_element_type=jnp.float32) * dv[1][:, None]


def _combine(out_ref, hcat_ref, dinv_ref, b0_ref, b1_ref, g_ref, beta_ref):
    dv = dinv_ref[...]
    sm = (out_ref[0] + hcat_ref[0]) * dv[0][:, None] + b0_ref[...]
    sm = sm + (out_ref[1] + hcat_ref[1]) * dv[1][:, None] + b1_ref[...]
    mu = jnp.mean(sm, axis=1, keepdims=True)
    var = jnp.mean((sm - mu) ** 2, axis=1, keepdims=True)
    hn = (sm - mu) * lax.rsqrt(var + 1e-5) * g_ref[...] + beta_ref[...]
    return jnp.maximum(hn, 0.0), dv


def _mid_body(out_ref, hcat_ref, dinv_ref, b0_ref, b1_ref, g_ref, beta_ref,
              w0_ref, w1_ref, hcat2_ref):
    h, dv = _combine(out_ref, hcat_ref, dinv_ref, b0_ref, b1_ref, g_ref, beta_ref)
    hcat2_ref[0] = jnp.dot(h, w0_ref[...],
                           preferred_element_type=jnp.float32) * dv[0][:, None]
    hcat2_ref[1] = jnp.dot(h, w1_ref[...],
                           preferred_element_type=jnp.float32) * dv[1][:, None]


def _final_body(out_ref, hcat_ref, dinv_ref, b0_ref, b1_ref, g_ref, beta_ref,
                h_ref):
    h, _ = _combine(out_ref, hcat_ref, dinv_ref, b0_ref, b1_ref, g_ref, beta_ref)
    h_ref[...] = h


_spec_nodes = pl.BlockSpec((_BM, D), lambda i: (i, 0))
_spec_w = pl.BlockSpec((D, D), lambda i: (0, 0))
_spec_vecD = pl.BlockSpec((1, D), lambda i: (0, 0))
_spec_2n = pl.BlockSpec((2, _BM), lambda i: (0, i))
_spec_2nd = pl.BlockSpec((2, _BM, D), lambda i: (0, i, 0))

_prep_call = pl.pallas_call(
    _prep_body,
    grid=(_GRID,),
    in_specs=[_spec_nodes, _spec_w, _spec_w, _spec_2n],
    out_specs=[_spec_2nd, _spec_2n],
    out_shape=[jax.ShapeDtypeStruct((2, NP, D), jnp.float32),
               jax.ShapeDtypeStruct((2, NP), jnp.float32)],
)

_mid_call = pl.pallas_call(
    _mid_body,
    grid=(_GRID,),
    in_specs=[_spec_2nd, _spec_2nd, _spec_2n, _spec_vecD, _spec_vecD,
              _spec_vecD, _spec_vecD, _spec_w, _spec_w],
    out_specs=_spec_2nd,
    out_shape=jax.ShapeDtypeStruct((2, NP, D), jnp.float32),
)

_final_call = pl.pallas_call(
    _final_body,
    grid=(_GRID,),
    in_specs=[_spec_2nd, _spec_2nd, _spec_2n, _spec_vecD, _spec_vecD,
              _spec_vecD, _spec_vecD],
    out_specs=_spec_nodes,
    out_shape=jax.ShapeDtypeStruct((NP, D), jnp.float32),
)


def kernel(x, edge_index_r0, edge_index_r1, W0_r0, b0_r0, W0_r1, b0_r1,
           W1_r0, b1_r0, W1_r1, b1_r1, ln0_g, ln0_b, ln1_g, ln1_b):
    f32 = jnp.float32
    src_all = jnp.concatenate(
        [edge_index_r0[0], edge_index_r1[0] + NP]).reshape(2 * NT, NCH, 1, K)
    dst_all = jnp.concatenate(
        [edge_index_r0[1], edge_index_r1[1]]).reshape(2 * NT, NCH, 1, K)
    ed_all = jnp.concatenate([src_all, dst_all], axis=2)
    zero1 = jnp.zeros((NP,), f32)
    zrows = jnp.zeros((NP, D), f32)
    x_pad = jnp.pad(x, ((0, NP - N), (0, 0)))

    _deg_kernel, _scatter_kernel = _sc_kernels()
    deg = _deg_kernel(dst_all.reshape(2 * NT, NCH_DEG, K_DEG), zero1)

    hcat1, dinv = _prep_call(x_pad, W0_r0, W0_r1, deg)
    out1 = _scatter_kernel(hcat1.reshape(2 * NP, D), ed_all, zrows)
    hcat2 = _mid_call(out1, hcat1, dinv,
                      b0_r0.reshape(1, D), b0_r1.reshape(1, D),
                      ln0_g.reshape(1, D), ln0_b.reshape(1, D), W1_r0, W1_r1)
    out2 = _scatter_kernel(hcat2.reshape(2 * NP, D), ed_all, zrows)
    h = _final_call(out2, hcat2, dinv,
                    b1_r0.reshape(1, D), b1_r1.reshape(1, D),
                    ln1_g.reshape(1, D), ln1_b.reshape(1, D))
    return h[:N]


# speedup vs baseline: 28.1268x; 1.1413x over previous
import functools

import jax
import jax.numpy as jnp
from jax import lax
from jax.experimental import pallas as pl
from jax.experimental.pallas import tpu as pltpu
from jax.experimental.pallas import tpu_sc as plsc

N = 10000
E = 160000
D = 128
NP = 10240
NT = 16
ROWS_PER_TILE = NP // NT
K = 125
NCH = (E // NT) // K
GB = 8
NGRP = NCH // GB
K_DEG = 80
NCH_DEG = (E // NT) // K_DEG


def _deg_kernel_body(dst_hbm, zero1_hbm, deg_hbm, dst_v, ones_v, hist_sh):
    c = lax.axis_index("c")
    s = lax.axis_index("s")
    wid = c * NT + s
    pltpu.sync_copy(dst_hbm.at[wid], dst_v)
    pltpu.sync_copy(zero1_hbm.at[pl.ds(s * ROWS_PER_TILE, ROWS_PER_TILE)],
                    hist_sh.at[pl.ds(s * ROWS_PER_TILE, ROWS_PER_TILE)])
    for i in range(K_DEG // 16):
        ones_v[pl.ds(i * 16, 16)] = jnp.ones((16,), jnp.float32)
    plsc.subcore_barrier()

    def body(j, carry):
        pltpu.sync_copy(ones_v, hist_sh.at[dst_v.at[j]], add=True)
        return carry

    lax.fori_loop(0, NCH_DEG, body, 0)
    plsc.subcore_barrier()
    pltpu.sync_copy(hist_sh.at[pl.ds(s * ROWS_PER_TILE, ROWS_PER_TILE)],
                    deg_hbm.at[c, pl.ds(s * ROWS_PER_TILE, ROWS_PER_TILE)])


def _scatter_kernel_body(hcat_hbm, ed_hbm, zrows_hbm, out_hbm,
                         idx_v, rows0, rows1, acc_sh, sem0, sem1, semi):
    c = lax.axis_index("c")
    s = lax.axis_index("s")
    wid = c * NT + s
    pltpu.sync_copy(zrows_hbm.at[pl.ds(s * ROWS_PER_TILE, ROWS_PER_TILE)],
                    acc_sh.at[pl.ds(s * ROWS_PER_TILE, ROWS_PER_TILE)])
    plsc.subcore_barrier()

    rows = (rows0, rows1)
    sems = (sem0, sem1)
    pltpu.sync_copy(ed_hbm.at[wid, 0], idx_v.at[0])
    pltpu.async_copy(hcat_hbm.at[idx_v.at[0, 0, 0]], rows0, sem0)

    def body(g, carry):
        pg = lax.rem(g, 2)
        png = 1 - pg

        @pl.when(g < NGRP - 1)
        def _prefetch():
            pltpu.async_copy(ed_hbm.at[wid, g + 1], idx_v.at[png], semi)

        for b in range(GB):
            rb, sb = rows[b % 2], sems[b % 2]
            if b < GB - 1:
                pltpu.async_copy(hcat_hbm.at[idx_v.at[pg, b + 1, 0]],
                                 rows[(b + 1) % 2], sems[(b + 1) % 2])
            else:
                @pl.when(g < NGRP - 1)
                def _next_group():
                    pltpu.make_async_copy(ed_hbm.at[wid, g + 1],
                                          idx_v.at[png], semi).wait()
                    pltpu.async_copy(hcat_hbm.at[idx_v.at[png, 0, 0]],
                                     rows[(b + 1) % 2], sems[(b + 1) % 2])
            pltpu.make_async_copy(hcat_hbm.at[idx_v.at[pg, b, 0]],
                                  rb, sb).wait()
            pltpu.sync_copy(rb, acc_sh.at[idx_v.at[pg, b, 1]], add=True)
        return carry

    lax.fori_loop(0, NGRP, body, 0)

    plsc.subcore_barrier()
    pltpu.sync_copy(acc_sh.at[pl.ds(s * ROWS_PER_TILE, ROWS_PER_TILE)],
                    out_hbm.at[c, pl.ds(s * ROWS_PER_TILE, ROWS_PER_TILE)])


@functools.lru_cache(maxsize=None)
def _sc_kernels():
    mesh = plsc.VectorSubcoreMesh(core_axis_name="c", subcore_axis_name="s")
    deg_k = pl.kernel(
        _deg_kernel_body,
        out_type=jax.ShapeDtypeStruct((2, NP), jnp.float32),
        mesh=mesh,
        scratch_types=[
            pltpu.VMEM((NCH_DEG, K_DEG), jnp.int32),
            pltpu.VMEM((K_DEG,), jnp.float32),
            pltpu.VMEM_SHARED((NP,), jnp.float32),
        ],
    )
    scatter_k = pl.kernel(
        _scatter_kernel_body,
        out_type=jax.ShapeDtypeStruct((2, NP, D), jnp.float32),
        mesh=mesh,
        scratch_types=[
            pltpu.VMEM((2, GB, 2, K), jnp.int32),
            pltpu.VMEM((K, D), jnp.float32),
            pltpu.VMEM((K, D), jnp.float32),
            pltpu.VMEM_SHARED((NP, D), jnp.float32),
            pltpu.SemaphoreType.DMA,
            pltpu.SemaphoreType.DMA,
            pltpu.SemaphoreType.DMA,
        ],
    )
    return deg_k, scatter_k


_BM = 1024
_GRID = NP // _BM


def _prep_body(x_ref, w0_ref, w1_ref, deg_ref, hcat_ref, dinv_ref):
    dv = lax.rsqrt(deg_ref[...] + 1.0)
    dinv_ref[...] = dv
    xb = x_ref[...]
    hcat_ref[0] = jnp.dot(xb, w0_ref[...],
                          preferred_element_type=jnp.float32) * dv[0][:, None]
    hcat_ref[1] = jnp.dot(xb, w1_ref[...],
                          preferred_element_type=jnp.float32) * dv[1][:, None]


def _combine(out_ref, hcat_ref, dinv_ref, b0_ref, b1_ref, g_ref, beta_ref):
    dv = dinv_ref[...]
    sm = (out_ref[0] + hcat_ref[0]) * dv[0][:, None] + b0_ref[...]
    sm = sm + (out_ref[1] + hcat_ref[1]) * dv[1][:, None] + b1_ref[...]
    mu = jnp.mean(sm, axis=1, keepdims=True)
    var = jnp.mean((sm - mu) ** 2, axis=1, keepdims=True)
    hn = (sm - mu) * lax.rsqrt(var + 1e-5) * g_ref[...] + beta_ref[...]
    return jnp.maximum(hn, 0.0), dv


def _mid_body(out_ref, hcat_ref, dinv_ref, b0_ref, b1_ref, g_ref, beta_ref,
              w0_ref, w1_ref, hcat2_ref):
    h, dv = _combine(out_ref, hcat_ref, dinv_ref, b0_ref, b1_ref, g_ref, beta_ref)
    hcat2_ref[0] = jnp.dot(h, w0_ref[...],
                           preferred_element_type=jnp.float32) * dv[0][:, None]
    hcat2_ref[1] = jnp.dot(h, w1_ref[...],
                           preferred_element_type=jnp.float32) * dv[1][:, None]


def _final_body(out_ref, hcat_ref, dinv_ref, b0_ref, b1_ref, g_ref, beta_ref,
                h_ref):
    h, _ = _combine(out_ref, hcat_ref, dinv_ref, b0_ref, b1_ref, g_ref, beta_ref)
    h_ref[...] = h


_spec_nodes = pl.BlockSpec((_BM, D), lambda i: (i, 0))
_spec_w = pl.BlockSpec((D, D), lambda i: (0, 0))
_spec_vecD = pl.BlockSpec((1, D), lambda i: (0, 0))
_spec_2n = pl.BlockSpec((2, _BM), lambda i: (0, i))
_spec_2nd = pl.BlockSpec((2, _BM, D), lambda i: (0, i, 0))

_prep_call = pl.pallas_call(
    _prep_body,
    grid=(_GRID,),
    in_specs=[_spec_nodes, _spec_w, _spec_w, _spec_2n],
    out_specs=[_spec_2nd, _spec_2n],
    out_shape=[jax.ShapeDtypeStruct((2, NP, D), jnp.float32),
               jax.ShapeDtypeStruct((2, NP), jnp.float32)],
)

_mid_call = pl.pallas_call(
    _mid_body,
    grid=(_GRID,),
    in_specs=[_spec_2nd, _spec_2nd, _spec_2n, _spec_vecD, _spec_vecD,
              _spec_vecD, _spec_vecD, _spec_w, _spec_w],
    out_specs=_spec_2nd,
    out_shape=jax.ShapeDtypeStruct((2, NP, D), jnp.float32),
)

_final_call = pl.pallas_call(
    _final_body,
    grid=(_GRID,),
    in_specs=[_spec_2nd, _spec_2nd, _spec_2n, _spec_vecD, _spec_vecD,
              _spec_vecD, _spec_vecD],
    out_specs=_spec_nodes,
    out_shape=jax.ShapeDtypeStruct((NP, D), jnp.float32),
)


def kernel(x, edge_index_r0, edge_index_r1, W0_r0, b0_r0, W0_r1, b0_r1,
           W1_r0, b1_r0, W1_r1, b1_r1, ln0_g, ln0_b, ln1_g, ln1_b):
    f32 = jnp.float32
    src_all = jnp.concatenate(
        [edge_index_r0[0], edge_index_r1[0] + NP]).reshape(2 * NT, NCH, 1, K)
    dst_all = jnp.concatenate(
        [edge_index_r0[1], edge_index_r1[1]]).reshape(2 * NT, NCH, 1, K)
    ed_all = jnp.concatenate([src_all, dst_all], axis=2).reshape(
        2 * NT, NGRP, GB, 2, K)
    zero1 = jnp.zeros((NP,), f32)
    zrows = jnp.zeros((NP, D), f32)
    x_pad = jnp.pad(x, ((0, NP - N), (0, 0)))

    _deg_kernel, _scatter_kernel = _sc_kernels()
    deg = _deg_kernel(dst_all.reshape(2 * NT, NCH_DEG, K_DEG), zero1)

    hcat1, dinv = _prep_call(x_pad, W0_r0, W0_r1, deg)
    out1 = _scatter_kernel(hcat1.reshape(2 * NP, D), ed_all, zrows)
    hcat2 = _mid_call(out1, hcat1, dinv,
                      b0_r0.reshape(1, D), b0_r1.reshape(1, D),
                      ln0_g.reshape(1, D), ln0_b.reshape(1, D), W1_r0, W1_r1)
    out2 = _scatter_kernel(hcat2.reshape(2 * NP, D), ed_all, zrows)
    h = _final_call(out2, hcat2, dinv,
                    b1_r0.reshape(1, D), b1_r1.reshape(1, D),
                    ln1_g.reshape(1, D), ln1_b.reshape(1, D))
    return h[:N]
